# baseline (speedup 1.0000x reference)
import functools

import jax
import jax.numpy as jnp
import numpy as _np
from jax import lax
from jax.experimental import pallas as pl
from jax.experimental.pallas import tpu as pltpu
from jax.experimental.pallas import tpu_sc as plsc

B = 16384
D = 16
L = 50
NF = 26
NC = 2
NS = 16
NW = NC * NS
BPW = B // NW
HCH = 64
NCH = BPW // HCH
EMB_W = (NF + 1) * D

_DIDX_PAT = _np.repeat(_np.arange(HCH, dtype=_np.int32), L)


def _sc_field(idx, tbl, out, idx_v, rows_v, sem):
    cid = lax.axis_index("c")
    sid = lax.axis_index("s")
    base = (cid * NS + sid) * BPW
    pltpu.sync_copy(idx.at[pl.ds(base, BPW)], idx_v)
    pltpu.async_copy(tbl.at[idx_v], rows_v, sem).wait()
    pltpu.sync_copy(rows_v, out.at[pl.ds(base, BPW), :])


def _make_field_call():
    return functools.partial(
        pl.kernel,
        out_type=jax.ShapeDtypeStruct((B, D), jnp.float32),
        mesh=plsc.VectorSubcoreMesh(core_axis_name="c",
                                    subcore_axis_name="s",
                                    num_cores=NC, num_subcores=NS),
        compiler_params=pltpu.CompilerParams(use_tc_tiling_on_sc=False),
        scratch_types=[
            pltpu.VMEM((BPW,), jnp.int32),
            pltpu.VMEM((BPW, D), jnp.float32),
            pltpu.SemaphoreType.DMA,
        ],
    )(_sc_field)


def _sc_hist(hist_flat, didx_pat, t_hist, out, hidx_v, hrows_v, didx_v,
             zero_v, accout_v, acc_sh, sem):
    cid = lax.axis_index("c")
    sid = lax.axis_index("s")
    base = (cid * NS + sid) * BPW

    pltpu.sync_copy(didx_pat, didx_v)
    woff = sid * HCH

    def _bd(i, c):
        sl = pl.ds(i * D, D)
        didx_v[sl] = didx_v[sl] + woff
        return c
    lax.fori_loop(0, HCH * L // D, _bd, 0)

    def _bz(i, c):
        zero_v[i, :] = jnp.zeros((D,), jnp.float32)
        return c
    lax.fori_loop(0, HCH, _bz, 0)

    for c in range(NCH):
        off = base * L + c * HCH * L
        pltpu.sync_copy(hist_flat.at[pl.ds(off, HCH * L)], hidx_v)
        pltpu.async_copy(t_hist.at[hidx_v], hrows_v, sem).wait()
        pltpu.sync_copy(zero_v, acc_sh.at[pl.ds(sid * HCH, HCH)])
        pltpu.sync_copy(hrows_v, acc_sh.at[didx_v], add=True)
        pltpu.sync_copy(acc_sh.at[pl.ds(sid * HCH, HCH)], accout_v)
        pltpu.sync_copy(accout_v, out.at[pl.ds(base + c * HCH, HCH), :])


_sc_hist_call = functools.partial(
    pl.kernel,
    out_type=jax.ShapeDtypeStruct((B, D), jnp.float32),
    mesh=plsc.VectorSubcoreMesh(core_axis_name="c", subcore_axis_name="s",
                                num_cores=NC, num_subcores=NS),
    compiler_params=pltpu.CompilerParams(use_tc_tiling_on_sc=False),
    scratch_types=[
        pltpu.VMEM((HCH * L,), jnp.int32),
        pltpu.VMEM((HCH * L, D), jnp.float32),
        pltpu.VMEM((HCH * L,), jnp.int32),
        pltpu.VMEM((HCH, D), jnp.float32),
        pltpu.VMEM((HCH, D), jnp.float32),
        pltpu.VMEM_SHARED((NS * HCH, D), jnp.float32),
        pltpu.SemaphoreType.DMA,
    ],
)(_sc_hist)


BB = 1024
GRID = B // BB


def _tc_body(emb_ref, price_ref, A_ref, w0_ref, wpe_ref, b0_ref, w1_ref,
             b1_ref, w2_ref, v_ref, scal_ref, out_ref):
    e = emb_ref[...]
    p = price_ref[...]
    w_lin = scal_ref[0:1, 0:1]
    b2 = scal_ref[0:1, 1:2]
    sv2 = scal_ref[0:1, 2:3]

    s = jnp.dot(e, A_ref[...], preferred_element_type=jnp.float32)
    s = s + p * v_ref[...]
    ss = jnp.sum(e * e, axis=1, keepdims=True) + p * p * sv2
    fm = 0.5 * (jnp.sum(s * s, axis=1, keepdims=True) - ss)

    h = jnp.dot(e, w0_ref[...], preferred_element_type=jnp.float32)
    h = jnp.maximum(h + p * wpe_ref[...] + b0_ref[...], 0.0)
    h = jnp.maximum(
        jnp.dot(h, w1_ref[...], preferred_element_type=jnp.float32)
        + b1_ref[...], 0.0)
    mlp = jnp.sum(h * w2_ref[...], axis=1, keepdims=True) + b2

    res = p * w_lin + fm + mlp
    out_ref[...] = (1.0 / (1.0 + jnp.exp(-res)))[:, 0]


def _tc_head(emb, price, A, w0, wpe, b0, w1, b1, w2, v, scal):
    full = lambda shape: pl.BlockSpec(shape, lambda i: (0, 0))
    return pl.pallas_call(
        _tc_body,
        grid=(GRID,),
        in_specs=[
            pl.BlockSpec((BB, EMB_W), lambda i: (i, 0)),
            pl.BlockSpec((BB, 1), lambda i: (i, 0)),
            full(A.shape), full(w0.shape), full(wpe.shape), full(b0.shape),
            full(w1.shape), full(b1.shape), full(w2.shape), full(v.shape),
            full(scal.shape),
        ],
        out_specs=pl.BlockSpec((BB,), lambda i: (i,)),
        out_shape=jax.ShapeDtypeStruct((B,), jnp.float32),
    )(emb, price, A, w0, wpe, b0, w1, b1, w2, v, scal)


def kernel(item_id, f01, f02, f03, f04, f05, f06, f07, f08, f09, f10, f11,
           f12, f13, f14, f15, f16, f17, f18, f19, f20, f21, f22, f23, f24,
           f25, hist, price, T_item_id, T_f01, T_f02, T_f03, T_f04, T_f05,
           T_f06, T_f07, T_f08, T_f09, T_f10, T_f11, T_f12, T_f13, T_f14,
           T_f15, T_f16, T_f17, T_f18, T_f19, T_f20, T_f21, T_f22, T_f23,
           T_f24, T_f25, T_hist, ctn_emb_price, ctn_lin_price, W0, b0, W1,
           b1, W2, b2):
    fields = (item_id, f01, f02, f03, f04, f05, f06, f07, f08, f09, f10,
              f11, f12, f13, f14, f15, f16, f17, f18, f19, f20, f21, f22,
              f23, f24, f25)
    tables = (T_item_id, T_f01, T_f02, T_f03, T_f04, T_f05, T_f06, T_f07,
              T_f08, T_f09, T_f10, T_f11, T_f12, T_f13, T_f14, T_f15,
              T_f16, T_f17, T_f18, T_f19, T_f20, T_f21, T_f22, T_f23,
              T_f24, T_f25)

    hist_flat = hist.astype(jnp.int32).reshape(B * L)
    didx_pat = jnp.asarray(_DIDX_PAT)

    slabs = [
        _make_field_call()(f.astype(jnp.int32).reshape(B), t)
        for f, t in zip(fields, tables)
    ]
    slabs.append(_sc_hist_call(hist_flat, didx_pat, T_hist))
    emb = jnp.concatenate(slabs, axis=1)

    v = ctn_emb_price.reshape(1, D).astype(jnp.float32)
    A = jnp.tile(jnp.eye(D, dtype=jnp.float32), (NF + 1, 1))
    w0_main = W0[:EMB_W, :]
    wpe = v @ W0[EMB_W:, :]
    scal = jnp.stack([ctn_lin_price[0, 0], b2[0],
                      jnp.sum(v * v)]).reshape(1, 3)
    return _tc_head(emb, price.astype(jnp.float32), A, w0_main, wpe,
                    b0.reshape(1, -1), W1, b1.reshape(1, -1),
                    W2.reshape(1, -1), v, scal)


# speedup vs baseline: 1.1622x; 1.1622x over previous
import functools

import jax
import jax.numpy as jnp
from jax import lax
from jax.experimental import pallas as pl
from jax.experimental.pallas import tpu as pltpu
from jax.experimental.pallas import tpu_sc as plsc

B = 16384
D = 16
L = 50
NF = 26
NC = 2
NS = 16
NW = NC * NS
BPW = B // NW
HCH = 64
NCH = BPW // HCH
EMB_W = (NF + 1) * D
NFM = 25
EMBM_W = (NFM + 1) * D

import numpy as _np
_DIDX_PAT = _np.repeat(_np.arange(HCH, dtype=_np.int32), L)


def _sc_field(idx, tbl, out, fidx_v, frows_v, fsem):
    cid = lax.axis_index("c")
    sid = lax.axis_index("s")
    base = (cid * NS + sid) * BPW
    pltpu.sync_copy(idx.at[pl.ds(base, BPW)], fidx_v)
    pltpu.async_copy(tbl.at[fidx_v], frows_v, fsem).wait()
    pltpu.sync_copy(frows_v, out.at[pl.ds(base, BPW), :])


_sc_item_call = functools.partial(
    pl.kernel,
    out_type=jax.ShapeDtypeStruct((B, D), jnp.float32),
    mesh=plsc.VectorSubcoreMesh(core_axis_name="c", subcore_axis_name="s",
                                num_cores=NC, num_subcores=NS),
    compiler_params=pltpu.CompilerParams(use_tc_tiling_on_sc=False),
    scratch_types=[
        pltpu.VMEM((BPW,), jnp.int32),
        pltpu.VMEM((BPW, D), jnp.float32),
        pltpu.SemaphoreType.DMA,
    ],
)(_sc_field)


def _sc_gather(*args):
    idxs = args[:NFM]
    hist_flat = args[NFM]
    didx_pat = args[NFM + 1]
    tbls = args[NFM + 2:2 * NFM + 2]
    t_hist = args[2 * NFM + 2]
    out = args[2 * NFM + 3]
    (idx_v, rows_a, rows_b, hidx_v, hrows_v, didx_v, zero_v, accout_v,
     acc_sh, sem_i, sem_a, sem_b) = args[2 * NFM + 4:]

    cid = lax.axis_index("c")
    sid = lax.axis_index("s")
    wid = cid * NS + sid
    base = wid * BPW

    idescs = [
        pltpu.async_copy(idxs[t].at[pl.ds(base, BPW)], idx_v.at[t], sem_i)
        for t in range(NFM)
    ]
    for dsc in idescs:
        dsc.wait()

    bufs = (rows_a, rows_b)
    sems = (sem_a, sem_b)
    descs = [None, None]
    descs[0] = pltpu.async_copy(tbls[0].at[idx_v.at[0]], bufs[0], sems[0])
    for t in range(NFM):
        if t + 1 < NFM:
            nb = (t + 1) % 2
            descs[nb] = pltpu.async_copy(
                tbls[t + 1].at[idx_v.at[t + 1]], bufs[nb], sems[nb])
        descs[t % 2].wait()
        pltpu.sync_copy(bufs[t % 2],
                        out.at[pl.ds(base, BPW), pl.ds(t * D, D)])

    pltpu.sync_copy(didx_pat, didx_v)
    off = sid * HCH

    def _bd(i, c):
        sl = pl.ds(i * D, D)
        didx_v[sl] = didx_v[sl] + off
        return c
    lax.fori_loop(0, HCH * L // D, _bd, 0)

    def _bz(i, c):
        zero_v[i, :] = jnp.zeros((D,), jnp.float32)
        return c
    lax.fori_loop(0, HCH, _bz, 0)

    for c in range(NCH):
        off = base * L + c * HCH * L
        pltpu.sync_copy(hist_flat.at[pl.ds(off, HCH * L)], hidx_v)
        pltpu.async_copy(t_hist.at[hidx_v], hrows_v, sem_a).wait()
        pltpu.sync_copy(zero_v, acc_sh.at[pl.ds(sid * HCH, HCH)])
        pltpu.sync_copy(hrows_v, acc_sh.at[didx_v], add=True)
        pltpu.sync_copy(acc_sh.at[pl.ds(sid * HCH, HCH)], accout_v)
        pltpu.sync_copy(
            accout_v,
            out.at[pl.ds(base + c * HCH, HCH), pl.ds(NFM * D, D)])


_sc_gather_call = functools.partial(
    pl.kernel,
    out_type=jax.ShapeDtypeStruct((B, EMBM_W), jnp.float32),
    mesh=plsc.VectorSubcoreMesh(core_axis_name="c", subcore_axis_name="s",
                                num_cores=NC, num_subcores=NS),
    compiler_params=pltpu.CompilerParams(use_tc_tiling_on_sc=False),
    scratch_types=[
        pltpu.VMEM((NFM, BPW), jnp.int32),
        pltpu.VMEM((BPW, D), jnp.float32),
        pltpu.VMEM((BPW, D), jnp.float32),
        pltpu.VMEM((HCH * L,), jnp.int32),
        pltpu.VMEM((HCH * L, D), jnp.float32),
        pltpu.VMEM((HCH * L,), jnp.int32),
        pltpu.VMEM((HCH, D), jnp.float32),
        pltpu.VMEM((HCH, D), jnp.float32),
        pltpu.VMEM_SHARED((NS * HCH, D), jnp.float32),
        pltpu.SemaphoreType.DMA,
        pltpu.SemaphoreType.DMA,
        pltpu.SemaphoreType.DMA,
    ],
)(_sc_gather)


BB = 1024
GRID = B // BB


def _tc_body(emb_ref, price_ref, A_ref, w0_ref, wpe_ref, b0_ref, w1_ref,
             b1_ref, w2_ref, v_ref, scal_ref, out_ref):
    e = emb_ref[...]
    p = price_ref[...]
    w_lin = scal_ref[0:1, 0:1]
    b2 = scal_ref[0:1, 1:2]
    sv2 = scal_ref[0:1, 2:3]

    s = jnp.dot(e, A_ref[...], preferred_element_type=jnp.float32)
    s = s + p * v_ref[...]
    ss = jnp.sum(e * e, axis=1, keepdims=True) + p * p * sv2
    fm = 0.5 * (jnp.sum(s * s, axis=1, keepdims=True) - ss)

    h = jnp.dot(e, w0_ref[...], preferred_element_type=jnp.float32)
    h = jnp.maximum(h + p * wpe_ref[...] + b0_ref[...], 0.0)
    h = jnp.maximum(
        jnp.dot(h, w1_ref[...], preferred_element_type=jnp.float32)
        + b1_ref[...], 0.0)
    mlp = jnp.sum(h * w2_ref[...], axis=1, keepdims=True) + b2

    res = p * w_lin + fm + mlp
    out_ref[...] = (1.0 / (1.0 + jnp.exp(-res)))[:, 0]


def _tc_head(emb, price, A, w0, wpe, b0, w1, b1, w2, v, scal):
    full = lambda shape: pl.BlockSpec(shape, lambda i: (0, 0))
    return pl.pallas_call(
        _tc_body,
        grid=(GRID,),
        in_specs=[
            pl.BlockSpec((BB, EMB_W), lambda i: (i, 0)),
            pl.BlockSpec((BB, 1), lambda i: (i, 0)),
            full(A.shape), full(w0.shape), full(wpe.shape), full(b0.shape),
            full(w1.shape), full(b1.shape), full(w2.shape), full(v.shape),
            full(scal.shape),
        ],
        out_specs=pl.BlockSpec((BB,), lambda i: (i,)),
        out_shape=jax.ShapeDtypeStruct((B,), jnp.float32),
    )(emb, price, A, w0, wpe, b0, w1, b1, w2, v, scal)


def kernel(item_id, f01, f02, f03, f04, f05, f06, f07, f08, f09, f10, f11,
           f12, f13, f14, f15, f16, f17, f18, f19, f20, f21, f22, f23, f24,
           f25, hist, price, T_item_id, T_f01, T_f02, T_f03, T_f04, T_f05,
           T_f06, T_f07, T_f08, T_f09, T_f10, T_f11, T_f12, T_f13, T_f14,
           T_f15, T_f16, T_f17, T_f18, T_f19, T_f20, T_f21, T_f22, T_f23,
           T_f24, T_f25, T_hist, ctn_emb_price, ctn_lin_price, W0, b0, W1,
           b1, W2, b2):
    fields = (item_id, f01, f02, f03, f04, f05, f06, f07, f08, f09, f10,
              f11, f12, f13, f14, f15, f16, f17, f18, f19, f20, f21, f22,
              f23, f24, f25)
    tables = (T_item_id, T_f01, T_f02, T_f03, T_f04, T_f05, T_f06, T_f07,
              T_f08, T_f09, T_f10, T_f11, T_f12, T_f13, T_f14, T_f15,
              T_f16, T_f17, T_f18, T_f19, T_f20, T_f21, T_f22, T_f23,
              T_f24, T_f25)

    idx_in = [f.astype(jnp.int32).reshape(B) for f in fields]
    hist_flat = hist.astype(jnp.int32).reshape(B * L)
    didx_pat = jnp.asarray(_DIDX_PAT)

    item_slab = _sc_item_call(idx_in[0], tables[0])
    rest = _sc_gather_call(*idx_in[1:], hist_flat, didx_pat, *tables[1:],
                           T_hist)
    emb = jnp.concatenate([item_slab, rest], axis=1)

    v = ctn_emb_price.reshape(1, D).astype(jnp.float32)
    A = jnp.tile(jnp.eye(D, dtype=jnp.float32), (NF + 1, 1))
    w0_main = W0[:EMB_W, :]
    wpe = v @ W0[EMB_W:, :]
    scal = jnp.stack([ctn_lin_price[0, 0], b2[0],
                      jnp.sum(v * v)]).reshape(1, 3)
    return _tc_head(emb, price.astype(jnp.float32), A, w0_main, wpe,
                    b0.reshape(1, -1), W1, b1.reshape(1, -1),
                    W2.reshape(1, -1), v, scal)
